# Initial kernel scaffold; baseline (speedup 1.0000x reference)
import jax
import jax.numpy as jnp
from jax import lax
from jax.experimental import pallas as pl
from jax.experimental.pallas import tpu as pltpu
from jax.experimental.pallas import tpu_sc as plsc

N = 10000
E = 320000
D = 128

NC = 2
NS = 16
L = 16
H = D // NC

EP = E // NS
K = 80
NCHUNK = EP // K

NP = N // NS
PC = 125
NPC = NP // PC


def _rsqrt_newton(d):
    bits = plsc.bitcast(d, jnp.int32)
    y = plsc.bitcast(jnp.int32(0x5F3759DF) - (bits >> 1), jnp.float32)
    for _ in range(3):
        y = y * (1.5 - 0.5 * d * y * y)
    return y


def _body(x_h, rows3, cols3, bias_h, out_h,
          xs_sp, acc_sp, degw_sp,
          rows2d, cols2d, stage0, stage1, ones_b, degw_l, dis_l,
          xstage, astage, bias_l, semg, sems):
    c = lax.axis_index("c")
    s = lax.axis_index("s")
    node0 = s * NP

    pltpu.sync_copy(rows3.at[s], rows2d)
    pltpu.sync_copy(cols3.at[s], cols2d)
    pltpu.sync_copy(bias_h.at[c], bias_l)

    ones16 = jnp.ones((L,), jnp.float32)
    zeros16 = jnp.zeros((L,), jnp.float32)

    def _fill_ones(i, _):
        ones_b[i] = ones16
        return 0
    lax.fori_loop(0, K, _fill_ones, 0)

    def _zero_degwl(i, _):
        degw_l[i] = zeros16
        return 0
    lax.fori_loop(0, PC, _zero_degwl, 0)

    def _zero_xstage(i, _):
        for v in range(H // L):
            xstage[i, pl.ds(v * L, L)] = zeros16
        return 0
    lax.fori_loop(0, PC, _zero_xstage, 0)

    for t in range(NPC):
        sl = pl.ds(node0 + t * PC, PC)
        pltpu.sync_copy(degw_l, degw_sp.at[sl])
        pltpu.sync_copy(xstage, acc_sp.at[sl])

    plsc.subcore_barrier()

    def _hist(j, _):
        pltpu.sync_copy(ones_b, degw_sp.at[cols2d.at[j]], add=True)
        return 0
    lax.fori_loop(0, NCHUNK, _hist, 0)

    plsc.subcore_barrier()

    for t in range(NPC):
        sl = pl.ds(node0 + t * PC, PC)
        pltpu.sync_copy(x_h.at[c, sl], xstage)
        pltpu.sync_copy(degw_sp.at[sl], degw_l)

        def _scale(i, _, t=t):
            d = degw_l[i] + 1.0
            y = _rsqrt_newton(d)
            dis_l[t * PC + i] = y
            for v in range(H // L):
                vsl = pl.ds(v * L, L)
                xstage[i, vsl] = xstage[i, vsl] * y
            return 0
        lax.fori_loop(0, PC, _scale, 0)

        pltpu.sync_copy(xstage, xs_sp.at[sl])

    plsc.subcore_barrier()

    pltpu.async_copy(xs_sp.at[rows2d.at[0]], stage0, semg).wait()

    def _edges(j, _):
        j0 = 2 * j
        g1 = pltpu.async_copy(xs_sp.at[rows2d.at[j0 + 1]], stage1, semg)
        s0 = pltpu.async_copy(stage0, acc_sp.at[cols2d.at[j0]], sems, add=True)
        g1.wait()
        s0.wait()
        s1 = pltpu.async_copy(stage1, acc_sp.at[cols2d.at[j0 + 1]], sems,
                              add=True)

        @pl.when(j < NCHUNK // 2 - 1)
        def _():
            pltpu.async_copy(xs_sp.at[rows2d.at[j0 + 2]], stage0, semg).wait()
        s1.wait()
        return 0
    lax.fori_loop(0, NCHUNK // 2, _edges, 0)

    plsc.subcore_barrier()

    for t in range(NPC):
        sl = pl.ds(node0 + t * PC, PC)
        pltpu.sync_copy(acc_sp.at[sl], astage)
        pltpu.sync_copy(xs_sp.at[sl], xstage)

        def _finish(i, _, t=t):
            y = dis_l[t * PC + i]
            for v in range(H // L):
                vsl = pl.ds(v * L, L)
                astage[i, vsl] = (astage[i, vsl] + xstage[i, vsl]) * y \
                    + bias_l[vsl]
            return 0
        lax.fori_loop(0, PC, _finish, 0)

        pltpu.sync_copy(astage, out_h.at[c, sl])


@jax.jit
def kernel(x, edge_index, bias):
    x_h = x.reshape(N, NC, H).transpose(1, 0, 2)
    rows3 = edge_index[0].reshape(NS, NCHUNK, K)
    cols3 = edge_index[1].reshape(NS, NCHUNK, K)
    bias_h = bias.reshape(NC, H)

    mesh = plsc.VectorSubcoreMesh(
        core_axis_name="c", subcore_axis_name="s",
        num_cores=NC, num_subcores=NS)

    run = pl.kernel(
        _body,
        out_type=jax.ShapeDtypeStruct((NC, N, H), jnp.float32),
        mesh=mesh,
        scratch_types=[
            pltpu.VMEM_SHARED((N, H), jnp.float32),
            pltpu.VMEM_SHARED((N, H), jnp.float32),
            pltpu.VMEM_SHARED((N, L), jnp.float32),
            pltpu.VMEM((NCHUNK, K), jnp.int32),
            pltpu.VMEM((NCHUNK, K), jnp.int32),
            pltpu.VMEM((K, H), jnp.float32),
            pltpu.VMEM((K, H), jnp.float32),
            pltpu.VMEM((K, L), jnp.float32),
            pltpu.VMEM((PC, L), jnp.float32),
            pltpu.VMEM((NP, L), jnp.float32),
            pltpu.VMEM((PC, H), jnp.float32),
            pltpu.VMEM((PC, H), jnp.float32),
            pltpu.VMEM((H,), jnp.float32),
            pltpu.SemaphoreType.DMA,
            pltpu.SemaphoreType.DMA,
        ],
    )
    out_h = run(x_h, rows3, cols3, bias_h)
    out = out_h.transpose(1, 0, 2).reshape(N, D)
    return (out, x)


# speedup vs baseline: 19.9694x; 19.9694x over previous
import functools

import jax
import jax.numpy as jnp
from jax import lax
from jax.experimental import pallas as pl
from jax.experimental.pallas import tpu as pltpu
from jax.experimental.pallas import tpu_sc as plsc

N = 10000
E = 320000
D = 128

NC = 2
NS = 16
NW = NC * NS
L = 16

EP = E // NW
K = 80
SCH = 25
NSUP = EP // (K * SCH)

NPAD = 10240
NP = NPAD // NS
PC = 32
NPC = NP // PC

TB = 1024


def _sc_kernels():
    mesh = plsc.VectorSubcoreMesh(
        core_axis_name="c", subcore_axis_name="s",
        num_cores=NC, num_subcores=NS)

    def _zero_acc(node0, zstage, acc_sp):
        zeros16 = jnp.zeros((L,), jnp.float32)

        def _zb(i, _):
            for v in range(D // L):
                zstage[i, pl.ds(v * L, L)] = zeros16
            return 0
        lax.fori_loop(0, PC, _zb, 0)

        def _zs(t, _):
            sl = pl.ds(pl.multiple_of(node0 + t * PC, PC), PC)
            pltpu.sync_copy(zstage, acc_sp.at[sl])
            return 0
        lax.fori_loop(0, NPC, _zs, 0)

    def _acc_out(c, node0, zstage, acc_sp, out_h):
        def _wb(t, _):
            sl = pl.ds(pl.multiple_of(node0 + t * PC, PC), PC)
            pltpu.sync_copy(acc_sp.at[sl], zstage)
            pltpu.sync_copy(zstage, out_h.at[c, sl])
            return 0
        lax.fori_loop(0, NPC, _wb, 0)

    @functools.partial(
        pl.kernel, mesh=mesh,
        out_type=jax.ShapeDtypeStruct((NC, NPAD, D), jnp.float32),
        scratch_types=[
            pltpu.VMEM_SHARED((NPAD, D), jnp.float32),
            pltpu.VMEM((SCH, K), jnp.int32),
            pltpu.VMEM((K, D), jnp.float32),
            pltpu.VMEM((PC, D), jnp.float32),
        ])
    def histogram(cols4, out_h, acc_sp, cols_sc, ones_b, zstage):
        c = lax.axis_index("c")
        s = lax.axis_index("s")
        wid = s * NC + c
        node0 = s * NP

        ones16 = jnp.ones((L,), jnp.float32)

        def _fo(i, _):
            for v in range(D // L):
                ones_b[i, pl.ds(v * L, L)] = ones16
            return 0
        lax.fori_loop(0, K, _fo, 0)
        _zero_acc(node0, zstage, acc_sp)
        plsc.subcore_barrier()

        def _sup(sup, _):
            pltpu.sync_copy(cols4.at[wid, sup], cols_sc)

            def _hist(j, _):
                pltpu.sync_copy(ones_b, acc_sp.at[cols_sc.at[j]], add=True)
                return 0
            lax.fori_loop(0, SCH, _hist, 0)
            return 0
        lax.fori_loop(0, NSUP, _sup, 0)
        plsc.subcore_barrier()
        _acc_out(c, node0, zstage, acc_sp, out_h)

    @functools.partial(
        pl.kernel, mesh=mesh,
        out_type=jax.ShapeDtypeStruct((NC, NPAD, D), jnp.float32),
        scratch_types=[
            pltpu.VMEM_SHARED((NPAD, D), jnp.float32),
            pltpu.VMEM((SCH, K), jnp.int32),
            pltpu.VMEM((SCH, K), jnp.int32),
            pltpu.VMEM((K, D), jnp.float32),
            pltpu.VMEM((PC, D), jnp.float32),
            pltpu.SemaphoreType.DMA,
        ])
    def propagate(xs_hbm, rows4, cols4, out_h,
                  acc_sp, rows_sc, cols_sc, stage0, zstage, semg):
        c = lax.axis_index("c")
        s = lax.axis_index("s")
        wid = s * NC + c
        node0 = s * NP

        _zero_acc(node0, zstage, acc_sp)
        plsc.subcore_barrier()

        def _sup(sup, _):
            pltpu.sync_copy(rows4.at[wid, sup], rows_sc)
            pltpu.sync_copy(cols4.at[wid, sup], cols_sc)

            def _edges(j, _):
                pltpu.async_copy(xs_hbm.at[rows_sc.at[j]], stage0,
                                 semg).wait()
                pltpu.sync_copy(stage0, acc_sp.at[cols_sc.at[j]], add=True)
                return 0
            lax.fori_loop(0, SCH, _edges, 0)
            return 0
        lax.fori_loop(0, NSUP, _sup, 0)
        plsc.subcore_barrier()
        _acc_out(c, node0, zstage, acc_sp, out_h)

    return histogram, propagate


_histogram, _propagate = _sc_kernels()


def _prescale_tc(p0, p1, x, o_xs, o_dis):
    dis = lax.rsqrt(p0[0] + p1[0] + 1.0)
    o_dis[...] = dis
    o_xs[...] = dis * x[...]


def _combine_tc(q0, q1, xs, dis, bias, o):
    o[...] = dis[...] * (q0[0] + q1[0] + xs[...]) + bias[...]


@jax.jit
def kernel(x, edge_index, bias):
    xpad = jnp.pad(x, ((0, NPAD - N), (0, 0)))
    rows4 = edge_index[0].reshape(NW, NSUP, SCH, K)
    cols4 = edge_index[1].reshape(NW, NSUP, SCH, K)

    deg_parts = _histogram(cols4)

    blk = pl.BlockSpec((TB, D), lambda i: (i, 0))
    xs, dis = pl.pallas_call(
        _prescale_tc,
        grid=(NPAD // TB,),
        in_specs=[pl.BlockSpec((1, TB, D), lambda i: (0, i, 0)),
                  pl.BlockSpec((1, TB, D), lambda i: (1, i, 0)),
                  blk],
        out_specs=[blk, blk],
        out_shape=[jax.ShapeDtypeStruct((NPAD, D), jnp.float32),
                   jax.ShapeDtypeStruct((NPAD, D), jnp.float32)],
    )(deg_parts, deg_parts, xpad)

    acc_parts = _propagate(xs, rows4, cols4)

    out = pl.pallas_call(
        _combine_tc,
        grid=(NPAD // TB,),
        in_specs=[pl.BlockSpec((1, TB, D), lambda i: (0, i, 0)),
                  pl.BlockSpec((1, TB, D), lambda i: (1, i, 0)),
                  blk, blk,
                  pl.BlockSpec((1, D), lambda i: (0, 0))],
        out_specs=blk,
        out_shape=jax.ShapeDtypeStruct((NPAD, D), jnp.float32),
    )(acc_parts, acc_parts, xs, dis, bias.reshape(1, D))

    return (out[:N], x)
